# baseline (speedup 1.0000x reference)
import functools
import jax
import jax.numpy as jnp
from jax import lax
from jax.experimental import pallas as pl
from jax.experimental.pallas import tpu as pltpu
from jax.experimental.pallas import tpu_sc as plsc

NUM_HEADS = 16
MAX_DISTANCE = 60
SEQ = 2048
A = 2040
LPAD = 4096


@functools.lru_cache(maxsize=1)
def _make_sc_kernel():
    mesh = plsc.VectorSubcoreMesh(
        core_axis_name="c", subcore_axis_name="s", num_cores=2, num_subcores=16
    )

    @functools.partial(
        pl.kernel,
        out_type=jax.ShapeDtypeStruct((NUM_HEADS * SEQ * SEQ,), jnp.float32),
        mesh=mesh,
        scratch_types=[
            pltpu.VMEM((8 * LPAD,), jnp.float32),
            pltpu.SemaphoreType.DMA,
        ],
    )
    def sc_kernel(w8_hbm, out_hbm, bank_v, sem):
        wid = lax.axis_index("s") * 2 + lax.axis_index("c")
        head = wid // 2
        half = wid % 2
        pltpu.sync_copy(w8_hbm.at[pl.ds(head * 8 * LPAD, 8 * LPAD)], bank_v)
        p0 = half * (SEQ // 2) // 8

        def group(g, carry):
            p = p0 + g
            off = A - 8 * p
            dst0 = (head * SEQ + 8 * p) * SEQ
            for s in range(8):
                pltpu.make_async_copy(
                    bank_v.at[pl.ds(s * LPAD + off, SEQ)],
                    out_hbm.at[pl.ds(dst0 + s * SEQ, SEQ)],
                    sem,
                ).start()
            for s in range(8):
                pltpu.make_async_copy(
                    bank_v.at[pl.ds(s * LPAD + off, SEQ)],
                    out_hbm.at[pl.ds(dst0 + s * SEQ, SEQ)],
                    sem,
                ).wait()
            return carry

        lax.fori_loop(0, (SEQ // 2) // 8, group, 0)

    return sc_kernel


@jax.jit
def kernel(seq_len, table):
    del seq_len
    m = jnp.arange(LPAD)
    s = jnp.arange(8)
    d = m[None, :] - s[:, None] - A
    idx = jnp.clip(d, -MAX_DISTANCE, MAX_DISTANCE) + MAX_DISTANCE
    w8 = jnp.transpose(table[idx], (2, 0, 1))
    out = _make_sc_kernel()(w8.reshape(-1))
    return out.reshape(NUM_HEADS, SEQ, SEQ)
